# baseline (speedup 1.0000x reference)
import functools

import jax
import jax.numpy as jnp
from jax import lax
from jax.experimental import pallas as pl
from jax.experimental.pallas import tpu as pltpu
from jax.experimental.pallas import tpu_sc as plsc

ROWS, COLS = 16384, 200
EMBED_DIM = 32

_info = plsc.get_sparse_core_info()
NC, NS = _info.num_cores, _info.num_subcores
NW = NC * NS
JH_N = COLS // 8
IB_N = ROWS // 128
SB_TOTAL = JH_N * IB_N
SB_PER_W = SB_TOTAL // NW
NOUTER = SB_PER_W // 2


@jax.jit
def _lookup(idx5, table):
    mesh = plsc.VectorSubcoreMesh(core_axis_name="c", subcore_axis_name="s")

    @functools.partial(
        pl.kernel,
        mesh=mesh,
        out_type=jax.ShapeDtypeStruct((COLS, 4, IB_N, 1024), jnp.float32),
        scratch_types=(
            [pltpu.VMEM((1024,), jnp.int32) for _ in range(2)]
            + [pltpu.VMEM((1024, EMBED_DIM), jnp.float32) for _ in range(2)]
            + [pltpu.VMEM((8, 4, 1024), jnp.float32)]
            + [pltpu.SemaphoreType.DMA for _ in range(5)]
        ),
        compiler_params=pltpu.CompilerParams(
            use_tc_tiling_on_sc=False, needs_layout_passes=False
        ),
    )
    def k(idx_hbm, table_hbm, out_hbm, i0, i1, g0, g1, tbuf, si0, si1, sg0, sg1, sw):
        idxb = (i0, i1)
        gbuf = (g0, g1)
        isem = (si0, si1)
        gsem = (sg0, sg1)

        wid = lax.axis_index("s") * NC + lax.axis_index("c")
        s0 = wid * SB_PER_W
        iota16 = lax.iota(jnp.int32, 16)

        def idx_slice(sb):
            return idx_hbm.at[sb >> 7, sb & 127]

        def out_slice(sb):
            return out_hbm.at[pl.ds((sb >> 7) * 8, 8), :, sb & 127]

        def start_idx(b, sb):
            pltpu.async_copy(idx_slice(sb), idxb[b], isem[b])

        def wait_idx(b, sb):
            pltpu.make_async_copy(idx_slice(sb), idxb[b], isem[b]).wait()

        def start_gather(b):
            pltpu.async_copy(table_hbm.at[idxb[b]], gbuf[b], gsem[b])

        def wait_gather(b):
            pltpu.make_async_copy(table_hbm.at[idxb[b]], gbuf[b], gsem[b]).wait()

        def start_write(sb):
            pltpu.async_copy(tbuf, out_slice(sb), sw)

        def wait_write(sb):
            pltpu.make_async_copy(tbuf, out_slice(sb), sw).wait()

        def transpose(b):
            g = gbuf[b]

            @plsc.parallel_loop(0, 8, unroll=2)
            def _(jl):
                jlv = jnp.full((16,), jl, jnp.int32)
                rows_g = [jl * 128 + gi * 16 + iota16 for gi in range(8)]
                il_g = [gi * 16 + iota16 for gi in range(8)]
                for d0 in range(32):
                    m = (d0 + iota16) & 31
                    dhv = m >> 3
                    tbase = (m & 7) << 7
                    for gi in range(8):
                        vec = plsc.load_gather(g, [rows_g[gi], m])
                        plsc.store_scatter(tbuf, [jlv, dhv, tbase + il_g[gi]], vec)

        start_idx(0, s0)
        wait_idx(0, s0)
        start_gather(0)
        start_idx(1, s0 + 1)

        def body(ko, carry):
            for b in (0, 1):
                sb = s0 + ko * 2 + b
                wait_gather(b)

                @pl.when(ko < NOUTER - 1)
                def _():
                    start_idx(b, sb + 2)

                if b == 0:

                    def _go():
                        wait_idx(1, sb + 1)
                        start_gather(1)

                    _go()
                else:

                    @pl.when(ko < NOUTER - 1)
                    def _():
                        wait_idx(0, sb + 1)
                        start_gather(0)

                if b == 0:

                    @pl.when(ko > 0)
                    def _():
                        wait_write(sb - 1)

                else:
                    wait_write(sb - 1)

                transpose(b)
                start_write(sb)
            return carry

        lax.fori_loop(0, NOUTER, body, 0)
        wait_write(s0 + SB_PER_W - 1)

    return k(idx5, table)


def kernel(edge_types, table):
    idx5 = (
        edge_types.astype(jnp.int32)
        .reshape(128, 128, JH_N, 8)
        .transpose(2, 0, 3, 1)
        .reshape(JH_N, IB_N, 1024)
    )
    out4 = _lookup(idx5, table)
    out5 = out4.reshape(COLS, 4, IB_N, 8, 128)
    return out5.transpose(2, 4, 0, 1, 3).reshape(ROWS, COLS, EMBED_DIM)


# speedup vs baseline: 1.3784x; 1.3784x over previous
import functools

import jax
import jax.numpy as jnp
from jax import lax
from jax.experimental import pallas as pl
from jax.experimental.pallas import tpu as pltpu
from jax.experimental.pallas import tpu_sc as plsc

ROWS, COLS = 16384, 200
EMBED_DIM = 32

_info = plsc.get_sparse_core_info()
NC, NS = _info.num_cores, _info.num_subcores
NW = NC * NS
JH_N = COLS // 8
IB_N = ROWS // 128
SB_TOTAL = JH_N * IB_N
SB_PER_W = SB_TOTAL // NW
NOUTER = SB_PER_W // 2


@jax.jit
def _lookup(idx5, table):
    mesh = plsc.VectorSubcoreMesh(core_axis_name="c", subcore_axis_name="s")

    @functools.partial(
        pl.kernel,
        mesh=mesh,
        out_type=jax.ShapeDtypeStruct((COLS * 4, IB_N, 1024), jnp.float32),
        scratch_types=(
            [pltpu.VMEM((1024,), jnp.int32) for _ in range(2)]
            + [pltpu.VMEM((1024, EMBED_DIM), jnp.float32) for _ in range(2)]
            + [pltpu.VMEM((32, 1024), jnp.float32)]
            + [pltpu.SemaphoreType.DMA for _ in range(5)]
        ),
        compiler_params=pltpu.CompilerParams(
            use_tc_tiling_on_sc=False, needs_layout_passes=False
        ),
    )
    def k(idx_hbm, table_hbm, out_hbm, i0, i1, g0, g1, tbuf, si0, si1, sg0, sg1, sw):
        idxb = (i0, i1)
        gbuf = (g0, g1)
        isem = (si0, si1)
        gsem = (sg0, sg1)

        wid = lax.axis_index("s") * NC + lax.axis_index("c")
        s0 = wid * SB_PER_W
        iota16 = lax.iota(jnp.int32, 16)

        def idx_slice(sb):
            return idx_hbm.at[sb >> 7, sb & 127]

        def out_slice(sb):
            return out_hbm.at[pl.ds((sb >> 7) * 32, 32), sb & 127]

        def start_idx(b, sb):
            pltpu.async_copy(idx_slice(sb), idxb[b], isem[b])

        def wait_idx(b, sb):
            pltpu.make_async_copy(idx_slice(sb), idxb[b], isem[b]).wait()

        def start_gather(b):
            pltpu.async_copy(table_hbm.at[idxb[b]], gbuf[b], gsem[b])

        def wait_gather(b):
            pltpu.make_async_copy(table_hbm.at[idxb[b]], gbuf[b], gsem[b]).wait()

        def start_write(sb):
            pltpu.async_copy(tbuf, out_slice(sb), sw)

        def wait_write(sb):
            pltpu.make_async_copy(tbuf, out_slice(sb), sw).wait()

        def transpose(b):
            g = gbuf[b]

            @plsc.parallel_loop(0, 8)
            def _(jl):
                jlv4 = jnp.full((16,), jl * 4, jnp.int32)
                rows_g = [jl * 128 + gi * 16 + iota16 for gi in range(8)]
                il_g = [gi * 16 + iota16 for gi in range(8)]
                for d0 in range(32):
                    m = (d0 + iota16) & 31
                    rowv = jlv4 + (m >> 3)
                    t0 = (m & 7) << 7
                    for gi in range(8):
                        vec = plsc.load_gather(g, [rows_g[gi], m])
                        plsc.store_scatter(tbuf, [rowv, t0 + il_g[gi]], vec)

        start_idx(0, s0)
        wait_idx(0, s0)
        start_gather(0)
        start_idx(1, s0 + 1)

        def body(ko, carry):
            for b in (0, 1):
                sb = s0 + ko * 2 + b
                wait_gather(b)

                @pl.when(ko < NOUTER - 1)
                def _():
                    start_idx(b, sb + 2)

                if b == 0:

                    def _go():
                        wait_idx(1, sb + 1)
                        start_gather(1)

                    _go()
                else:

                    @pl.when(ko < NOUTER - 1)
                    def _():
                        wait_idx(0, sb + 1)
                        start_gather(0)

                if b == 0:

                    @pl.when(ko > 0)
                    def _():
                        wait_write(sb - 1)

                else:
                    wait_write(sb - 1)

                transpose(b)
                start_write(sb)
            return carry

        lax.fori_loop(0, NOUTER, body, 0)
        wait_write(s0 + SB_PER_W - 1)

    return k(idx5, table)


def kernel(edge_types, table):
    idx5 = (
        edge_types.astype(jnp.int32)
        .reshape(128, 128, JH_N, 8)
        .transpose(2, 0, 3, 1)
        .reshape(JH_N, IB_N, 1024)
    )
    out4 = _lookup(idx5, table)
    out5 = out4.reshape(COLS, 4, IB_N, 8, 128)
    return out5.transpose(2, 4, 0, 1, 3).reshape(ROWS, COLS, EMBED_DIM)


# speedup vs baseline: 1.7491x; 1.2690x over previous
import functools

import jax
import jax.numpy as jnp
from jax import lax
from jax.experimental import pallas as pl
from jax.experimental.pallas import tpu as pltpu
from jax.experimental.pallas import tpu_sc as plsc

ROWS, COLS = 16384, 200
EMBED_DIM = 32

_info = plsc.get_sparse_core_info()
NC, NS = _info.num_cores, _info.num_subcores
NW = NC * NS
JH_N = COLS // 8
IB_N = ROWS // 128
U_TOTAL = JH_N * IB_N * 2
U_PER_W = U_TOTAL // NW
NOUTER = U_PER_W // 2


@jax.jit
def _lookup(idx5, table):
    mesh = plsc.VectorSubcoreMesh(core_axis_name="c", subcore_axis_name="s")

    @functools.partial(
        pl.kernel,
        mesh=mesh,
        out_type=jax.ShapeDtypeStruct((COLS * 4, IB_N, 1024), jnp.float32),
        scratch_types=(
            [pltpu.VMEM((512,), jnp.int32) for _ in range(2)]
            + [pltpu.VMEM((512, EMBED_DIM), jnp.float32) for _ in range(2)]
            + [pltpu.VMEM((16, 1024), jnp.float32) for _ in range(2)]
            + [pltpu.SemaphoreType.DMA for _ in range(6)]
        ),
        compiler_params=pltpu.CompilerParams(
            use_tc_tiling_on_sc=False, needs_layout_passes=False
        ),
    )
    def k(
        idx_hbm, table_hbm, out_hbm, i0, i1, g0, g1, t0_, t1_,
        si0, si1, sg0, sg1, sw0, sw1,
    ):
        idxb = (i0, i1)
        gbuf = (g0, g1)
        tb = (t0_, t1_)
        isem = (si0, si1)
        gsem = (sg0, sg1)
        wsem = (sw0, sw1)

        wid = lax.axis_index("s") * NC + lax.axis_index("c")
        s0 = wid * U_PER_W
        iota16 = lax.iota(jnp.int32, 16)

        def idx_slice(u):
            return idx_hbm.at[u >> 8, (u >> 1) & 127, pl.ds((u & 1) * 512, 512)]

        def out_slice(u):
            return out_hbm.at[
                pl.ds((u >> 8) * 32 + (u & 1) * 16, 16), (u >> 1) & 127
            ]

        def start_idx(b, u):
            pltpu.async_copy(idx_slice(u), idxb[b], isem[b])

        def wait_idx(b, u):
            pltpu.make_async_copy(idx_slice(u), idxb[b], isem[b]).wait()

        def start_gather(b):
            pltpu.async_copy(table_hbm.at[idxb[b]], gbuf[b], gsem[b])

        def wait_gather(b):
            pltpu.make_async_copy(table_hbm.at[idxb[b]], gbuf[b], gsem[b]).wait()

        def start_write(b, u):
            pltpu.async_copy(tb[b], out_slice(u), wsem[b])

        def wait_write(b, u):
            pltpu.make_async_copy(tb[b], out_slice(u), wsem[b]).wait()

        def transpose(b):
            g = gbuf[b]
            t = tb[b]

            @plsc.parallel_loop(0, 4)
            def _(jl):
                jlv4 = jnp.full((16,), jl * 4, jnp.int32)
                rows_g = [jl * 128 + gi * 16 + iota16 for gi in range(8)]
                il_g = [gi * 16 + iota16 for gi in range(8)]
                for d0 in range(32):
                    m = (d0 + iota16) & 31
                    rowv = jlv4 + (m >> 3)
                    tcol = (m & 7) << 7
                    for gi in range(8):
                        vec = plsc.load_gather(g, [rows_g[gi], m])
                        plsc.store_scatter(t, [rowv, tcol + il_g[gi]], vec)

        start_idx(0, s0)
        wait_idx(0, s0)
        start_gather(0)
        start_idx(1, s0 + 1)

        def body(ko, carry):
            for b in (0, 1):
                u = s0 + ko * 2 + b
                wait_gather(b)

                @pl.when(ko < NOUTER - 1)
                def _():
                    start_idx(b, u + 2)

                if b == 0:

                    def _go():
                        wait_idx(1, u + 1)
                        start_gather(1)

                    _go()
                else:

                    @pl.when(ko < NOUTER - 1)
                    def _():
                        wait_idx(0, u + 1)
                        start_gather(0)

                @pl.when(ko > 0)
                def _():
                    wait_write(b, u - 2)

                transpose(b)
                start_write(b, u)
            return carry

        lax.fori_loop(0, NOUTER, body, 0)
        wait_write(0, s0 + U_PER_W - 2)
        wait_write(1, s0 + U_PER_W - 1)

    return k(idx5, table)


def kernel(edge_types, table):
    idx5 = (
        edge_types.astype(jnp.int32)
        .reshape(128, 128, JH_N, 8)
        .transpose(2, 0, 3, 1)
        .reshape(JH_N, IB_N, 1024)
    )
    out4 = _lookup(idx5, table)
    out5 = out4.reshape(COLS, 4, IB_N, 8, 128)
    return out5.transpose(2, 4, 0, 1, 3).reshape(ROWS, COLS, EMBED_DIM)
